# baseline (speedup 1.0000x reference)
import functools
import numpy as np
import jax
import jax.numpy as jnp
from jax import lax
from jax.experimental import pallas as pl
from jax.experimental.pallas import tpu as pltpu
from jax.experimental.pallas import tpu_sc as plsc

UNIGRAM_VOCAB = 1024
NGRAM_VOCAB = 768 * 256
NUM_HEADS = 16
DIM_PER_HEAD = 64
NGRAM_EMB_DIM = 8
EPS = 1e-05

B = 4
N = 2048
POS = B * N
MODEL = NUM_HEADS * DIM_PER_HEAD
NGFLAT = NUM_HEADS * NGRAM_EMB_DIM

NW = 32
POS_W = POS // NW
IDS_W = POS_W * NUM_HEADS
CHUNK = 128
NCHUNK = IDS_W // CHUNK


def _hash_primes(n, count):
    primes = []
    c = n + 1
    while len(primes) < count:
        k, is_p = 2, True
        while k * k <= c:
            if c % k == 0:
                is_p = False
                break
            k += 1
        if is_p:
            primes.append(c)
        c += 1
    return primes


PRIMES_I32 = np.array(_hash_primes(NGRAM_VOCAB, NUM_HEADS), dtype=np.int32)


_sc_mesh = plsc.VectorSubcoreMesh(core_axis_name="c", subcore_axis_name="s")


@functools.partial(
    pl.kernel,
    out_type=jax.ShapeDtypeStruct((NW * NCHUNK, CHUNK, NGRAM_EMB_DIM), jnp.float32),
    mesh=_sc_mesh,
    compiler_params=pltpu.CompilerParams(
        use_tc_tiling_on_sc=False, needs_layout_passes=False),
    scratch_types=[
        pltpu.VMEM((POS_W + 8,), jnp.int32),
        pltpu.VMEM((NCHUNK, CHUNK), jnp.int32),
        pltpu.VMEM((NCHUNK, CHUNK, NGRAM_EMB_DIM), jnp.float32),
        pltpu.SemaphoreType.DMA,
    ],
)
def _sc_gather(table_hbm, cids_hbm, out_hbm, buf_v, ids_v, rows_v, sem):
    cid = lax.axis_index("c")
    sid = lax.axis_index("s")
    wid = sid * 2 + cid
    base = wid * POS_W

    @pl.when(wid > 0)
    def _():
        pltpu.sync_copy(cids_hbm.at[pl.ds(base - 8, POS_W + 8)], buf_v)

    @pl.when(wid == 0)
    def _():
        pltpu.sync_copy(cids_hbm.at[pl.ds(0, POS_W)], buf_v.at[pl.ds(8, POS_W)])

    lanes = lax.iota(jnp.int32, 16)
    seq_start = base % N == 0

    def compute_ids(g, carry):
        cur = buf_v[pl.ds(8 + g * 16, 16)]
        prev = buf_v[pl.ds(7 + g * 16, 16)]
        first = jnp.logical_and(
            jnp.full((16,), jnp.logical_and(seq_start, g == 0)), lanes == 0)
        prev = jnp.where(first, 0, prev)
        pair = cur + prev * UNIGRAM_VOCAB
        fbase = (g * 16 + lanes) * NUM_HEADS
        for h in range(NUM_HEADS):
            ids = (pair * (h + 1) + (h + 1)) % int(PRIMES_I32[h])
            ids = ids % NGRAM_VOCAB + h * NGRAM_VOCAB
            fi = fbase + h
            plsc.store_scatter(ids_v, [fi // CHUNK, fi % CHUNK], ids)
        return carry

    lax.fori_loop(0, POS_W // 16, compute_ids, 0)

    def start(j):
        return pltpu.make_async_copy(
            table_hbm.at[ids_v.at[j]], rows_v.at[j], sem)

    start(0).start()

    def gather_chunk(j, carry):
        @pl.when(j < NCHUNK - 1)
        def _():
            start(j + 1).start()
        start(j).wait()
        return carry

    lax.fori_loop(0, NCHUNK, gather_chunk, 0)

    pltpu.sync_copy(rows_v, out_hbm.at[pl.ds(wid * NCHUNK, NCHUNK)])


_heads64 = np.arange(MODEL) // DIM_PER_HEAD
S64 = (_heads64[:, None] == np.arange(NUM_HEADS)[None, :]).astype(np.float32)
S64T = np.ascontiguousarray(S64.T)
_heads8 = np.arange(NGFLAT) // NGRAM_EMB_DIM
S8 = (_heads8[:, None] == np.arange(NUM_HEADS)[None, :]).astype(np.float32)
S8T = np.ascontiguousarray(S8.T)
PLACE = np.zeros((NGFLAT, MODEL), dtype=np.float32)
for _j in range(NGFLAT):
    PLACE[_j, (_j // NGRAM_EMB_DIM) * DIM_PER_HEAD +
          (DIM_PER_HEAD - NGRAM_EMB_DIM) + (_j % NGRAM_EMB_DIM)] = 1.0
KEEP = ((np.arange(MODEL) % DIM_PER_HEAD < DIM_PER_HEAD - NGRAM_EMB_DIM)
        .astype(np.float32)[None, :])

ROWS_BLK = 512
HP = jax.lax.Precision.HIGHEST


def _tc_body(x_ref, ng_ref, s64_ref, s64t_ref, s8_ref, s8t_ref, p_ref,
             ge_ref, be_ref, gn_ref, bn_ref, o_ref):
    x = x_ref[...]
    s64 = s64_ref[...]
    s64t = s64t_ref[...]
    mean = lax.dot(x, s64, precision=HP) * (1.0 / DIM_PER_HEAD)
    xc = x - lax.dot(mean, s64t, precision=HP)
    var = lax.dot(xc * xc, s64, precision=HP) * (1.0 / DIM_PER_HEAD)
    inv = 1.0 / (jnp.sqrt(var) + EPS)
    part = xc * lax.dot(inv, s64t, precision=HP) * ge_ref[...] + be_ref[...]

    ng = ng_ref[...]
    s8 = s8_ref[...]
    s8t = s8t_ref[...]
    m8 = lax.dot(ng, s8, precision=HP) * (1.0 / NGRAM_EMB_DIM)
    nc = ng - lax.dot(m8, s8t, precision=HP)
    v8 = lax.dot(nc * nc, s8, precision=HP) * (1.0 / NGRAM_EMB_DIM)
    inv8 = 1.0 / (jnp.sqrt(v8) + EPS)
    nn = nc * lax.dot(inv8, s8t, precision=HP) * gn_ref[...] + bn_ref[...]

    o_ref[...] = part + lax.dot(nn, p_ref[...])


_tc_norm = pl.pallas_call(
    _tc_body,
    grid=(POS // ROWS_BLK,),
    in_specs=[
        pl.BlockSpec((ROWS_BLK, MODEL), lambda i: (i, 0)),
        pl.BlockSpec((ROWS_BLK, NGFLAT), lambda i: (i, 0)),
        pl.BlockSpec((MODEL, NUM_HEADS), lambda i: (0, 0)),
        pl.BlockSpec((NUM_HEADS, MODEL), lambda i: (0, 0)),
        pl.BlockSpec((NGFLAT, NUM_HEADS), lambda i: (0, 0)),
        pl.BlockSpec((NUM_HEADS, NGFLAT), lambda i: (0, 0)),
        pl.BlockSpec((NGFLAT, MODEL), lambda i: (0, 0)),
        pl.BlockSpec((1, MODEL), lambda i: (0, 0)),
        pl.BlockSpec((1, MODEL), lambda i: (0, 0)),
        pl.BlockSpec((1, NGFLAT), lambda i: (0, 0)),
        pl.BlockSpec((1, NGFLAT), lambda i: (0, 0)),
    ],
    out_specs=pl.BlockSpec((ROWS_BLK, MODEL), lambda i: (i, 0)),
    out_shape=jax.ShapeDtypeStruct((POS, MODEL), jnp.float32),
)


def kernel(embeds, cluster_ids, table, g_ngram, b_ngram, g_emb, b_emb):
    b, n, d = embeds.shape
    cids = cluster_ids.reshape(-1).astype(jnp.int32)
    gathered = _sc_gather(table, cids)
    ng = gathered.reshape(POS, NGFLAT)
    x = embeds.reshape(POS, MODEL)
    ge = g_emb.reshape(1, MODEL) * KEEP
    be = b_emb.reshape(1, MODEL) * KEEP
    gn = g_ngram.reshape(1, NGFLAT)
    bn = b_ngram.reshape(1, NGFLAT)
    f32 = jnp.float32
    out = _tc_norm(x, ng, jnp.asarray(S64, f32), jnp.asarray(S64T, f32),
                   jnp.asarray(S8, f32), jnp.asarray(S8T, f32),
                   jnp.asarray(PLACE, f32), ge, be, gn, bn)
    return out.reshape(b, n, d)


# speedup vs baseline: 3.6432x; 3.6432x over previous
import functools
import numpy as np
import jax
import jax.numpy as jnp
from jax import lax
from jax.experimental import pallas as pl
from jax.experimental.pallas import tpu as pltpu
from jax.experimental.pallas import tpu_sc as plsc

UNIGRAM_VOCAB = 1024
NGRAM_VOCAB = 768 * 256
NUM_HEADS = 16
DIM_PER_HEAD = 64
NGRAM_EMB_DIM = 8
EPS = 1e-05

B = 4
N = 2048
POS = B * N
MODEL = NUM_HEADS * DIM_PER_HEAD
NGFLAT = NUM_HEADS * NGRAM_EMB_DIM

NW = 32
POS_W = POS // NW
CHUNK = NGFLAT


def _hash_primes(n, count):
    primes = []
    c = n + 1
    while len(primes) < count:
        k, is_p = 2, True
        while k * k <= c:
            if c % k == 0:
                is_p = False
                break
            k += 1
        if is_p:
            primes.append(c)
        c += 1
    return primes


PRIMES_I32 = np.array(_hash_primes(NGRAM_VOCAB, NUM_HEADS), dtype=np.int32)


_sc_mesh = plsc.VectorSubcoreMesh(core_axis_name="c", subcore_axis_name="s")


@functools.partial(
    pl.kernel,
    out_type=jax.ShapeDtypeStruct((POS, NGFLAT), jnp.float32),
    mesh=_sc_mesh,
    compiler_params=pltpu.CompilerParams(
        use_tc_tiling_on_sc=False, needs_layout_passes=False),
    scratch_types=[
        pltpu.VMEM((POS_W + 8,), jnp.int32),
        pltpu.VMEM((POS_W, CHUNK), jnp.int32),
        pltpu.VMEM((POS_W, CHUNK), jnp.float32),
        pltpu.SemaphoreType.DMA,
    ],
)
def _sc_gather(tblf_hbm, cids_hbm, out_hbm, buf_v, ids_v, rows_v, sem):
    cid = lax.axis_index("c")
    sid = lax.axis_index("s")
    wid = sid * 2 + cid
    base = wid * POS_W

    @pl.when(wid > 0)
    def _():
        pltpu.sync_copy(cids_hbm.at[pl.ds(base - 8, POS_W + 8)], buf_v)

    @pl.when(wid == 0)
    def _():
        pltpu.sync_copy(cids_hbm.at[pl.ds(0, POS_W)], buf_v.at[pl.ds(8, POS_W)])

    seq_start = base % N == 0
    lane = lax.iota(jnp.int32, 16)
    half = lane >> 3
    d_off = (lane & 7) * 128
    mv, pv_, cv = [], [], []
    for v in range(8):
        h = 2 * v + half
        mv.append(h + 1)
        pv_.append(jnp.where(half == 0, int(PRIMES_I32[2 * v]),
                             int(PRIMES_I32[2 * v + 1])))
        cv.append(h * (1536 * 1024) + d_off)

    def compute(p, carry):
        cur = plsc.load_gather(buf_v, [jnp.full((16,), 8, jnp.int32) + p])
        prev = plsc.load_gather(buf_v, [jnp.full((16,), 7, jnp.int32) + p])
        prev = jnp.where(
            jnp.full((16,), jnp.logical_and(seq_start, p == 0)), 0, prev)
        pair = cur + prev * UNIGRAM_VOCAB
        for v in range(8):
            b = (pair * mv[v] + mv[v]) % pv_[v]
            b = jnp.where(b >= NGRAM_VOCAB, b - NGRAM_VOCAB, b)
            w = ((b >> 7) << 10) + (b & 127)
            ids_v[p, pl.ds(v * 16, 16)] = w + cv[v]
        return carry

    lax.fori_loop(0, POS_W, compute, 0)

    def start(j):
        return pltpu.make_async_copy(
            tblf_hbm.at[ids_v.at[j]], rows_v.at[j], sem)

    start(0).start()
    start(1).start()
    start(2).start()

    def gather_chunk(j, carry):
        @pl.when(j < POS_W - 3)
        def _():
            start(j + 3).start()
        start(j).wait()
        return carry

    lax.fori_loop(0, POS_W, gather_chunk, 0)

    pltpu.sync_copy(rows_v, out_hbm.at[pl.ds(base, POS_W)])


_heads64 = np.arange(MODEL) // DIM_PER_HEAD
S64 = (_heads64[:, None] == np.arange(NUM_HEADS)[None, :]).astype(np.float32)
S64T = np.ascontiguousarray(S64.T)
_heads8 = np.arange(NGFLAT) // NGRAM_EMB_DIM
S8 = (_heads8[:, None] == np.arange(NUM_HEADS)[None, :]).astype(np.float32)
S8T = np.ascontiguousarray(S8.T)
PLACE = np.zeros((NGFLAT, MODEL), dtype=np.float32)
for _j in range(NGFLAT):
    PLACE[_j, (_j // NGRAM_EMB_DIM) * DIM_PER_HEAD +
          (DIM_PER_HEAD - NGRAM_EMB_DIM) + (_j % NGRAM_EMB_DIM)] = 1.0
KEEP = ((np.arange(MODEL) % DIM_PER_HEAD < DIM_PER_HEAD - NGRAM_EMB_DIM)
        .astype(np.float32)[None, :])

ROWS_BLK = 512
HP = jax.lax.Precision.HIGHEST


def _tc_body(x_ref, ng_ref, s64_ref, s64t_ref, s8_ref, s8t_ref, p_ref,
             ge_ref, be_ref, gn_ref, bn_ref, o_ref):
    x = x_ref[...]
    s64 = s64_ref[...]
    s64t = s64t_ref[...]
    mean = lax.dot(x, s64, precision=HP) * (1.0 / DIM_PER_HEAD)
    xc = x - lax.dot(mean, s64t, precision=HP)
    var = lax.dot(xc * xc, s64, precision=HP) * (1.0 / DIM_PER_HEAD)
    inv = 1.0 / (jnp.sqrt(var) + EPS)
    part = xc * lax.dot(inv, s64t, precision=HP) * ge_ref[...] + be_ref[...]

    ng = ng_ref[...]
    s8 = s8_ref[...]
    s8t = s8t_ref[...]
    m8 = lax.dot(ng, s8, precision=HP) * (1.0 / NGRAM_EMB_DIM)
    nc = ng - lax.dot(m8, s8t, precision=HP)
    v8 = lax.dot(nc * nc, s8, precision=HP) * (1.0 / NGRAM_EMB_DIM)
    inv8 = 1.0 / (jnp.sqrt(v8) + EPS)
    nn = nc * lax.dot(inv8, s8t, precision=HP) * gn_ref[...] + bn_ref[...]

    o_ref[...] = part + lax.dot(nn, p_ref[...])


_tc_norm = pl.pallas_call(
    _tc_body,
    grid=(POS // ROWS_BLK,),
    in_specs=[
        pl.BlockSpec((ROWS_BLK, MODEL), lambda i: (i, 0)),
        pl.BlockSpec((ROWS_BLK, NGFLAT), lambda i: (i, 0)),
        pl.BlockSpec((MODEL, NUM_HEADS), lambda i: (0, 0)),
        pl.BlockSpec((NUM_HEADS, MODEL), lambda i: (0, 0)),
        pl.BlockSpec((NGFLAT, NUM_HEADS), lambda i: (0, 0)),
        pl.BlockSpec((NUM_HEADS, NGFLAT), lambda i: (0, 0)),
        pl.BlockSpec((NGFLAT, MODEL), lambda i: (0, 0)),
        pl.BlockSpec((1, MODEL), lambda i: (0, 0)),
        pl.BlockSpec((1, MODEL), lambda i: (0, 0)),
        pl.BlockSpec((1, NGFLAT), lambda i: (0, 0)),
        pl.BlockSpec((1, NGFLAT), lambda i: (0, 0)),
    ],
    out_specs=pl.BlockSpec((ROWS_BLK, MODEL), lambda i: (i, 0)),
    out_shape=jax.ShapeDtypeStruct((POS, MODEL), jnp.float32),
)


def kernel(embeds, cluster_ids, table, g_ngram, b_ngram, g_emb, b_emb):
    b, n, d = embeds.shape
    cids = cluster_ids.reshape(-1).astype(jnp.int32)
    tblf = table.T.reshape(NGRAM_EMB_DIM, -1, 128).transpose(1, 0, 2).reshape(-1)
    ng = _sc_gather(tblf, cids)
    x = embeds.reshape(POS, MODEL)
    ge = g_emb.reshape(1, MODEL) * KEEP
    be = b_emb.reshape(1, MODEL) * KEEP
    gn = g_ngram.reshape(1, NGFLAT)
    bn = b_ngram.reshape(1, NGFLAT)
    f32 = jnp.float32
    out = _tc_norm(x, ng, jnp.asarray(S64, f32), jnp.asarray(S64T, f32),
                   jnp.asarray(S8, f32), jnp.asarray(S8T, f32),
                   jnp.asarray(PLACE, f32), ge, be, gn, bn)
    return out.reshape(b, n, d)


# speedup vs baseline: 6.6692x; 1.8306x over previous
import functools
import numpy as np
import jax
import jax.numpy as jnp
from jax import lax
from jax.experimental import pallas as pl
from jax.experimental.pallas import tpu as pltpu
from jax.experimental.pallas import tpu_sc as plsc

UNIGRAM_VOCAB = 1024
NGRAM_VOCAB = 768 * 256
NUM_HEADS = 16
DIM_PER_HEAD = 64
NGRAM_EMB_DIM = 8
EPS = 1e-05

B = 4
N = 2048
POS = B * N
MODEL = NUM_HEADS * DIM_PER_HEAD
NGFLAT = NUM_HEADS * NGRAM_EMB_DIM

NW = 32
POS_W = POS // NW
CHUNK = NGFLAT


def _hash_primes(n, count):
    primes = []
    c = n + 1
    while len(primes) < count:
        k, is_p = 2, True
        while k * k <= c:
            if c % k == 0:
                is_p = False
                break
            k += 1
        if is_p:
            primes.append(c)
        c += 1
    return primes


PRIMES_I32 = np.array(_hash_primes(NGRAM_VOCAB, NUM_HEADS), dtype=np.int32)


_sc_mesh = plsc.VectorSubcoreMesh(core_axis_name="c", subcore_axis_name="s")


@functools.partial(
    pl.kernel,
    out_type=jax.ShapeDtypeStruct((POS, NGFLAT), jnp.float32),
    mesh=_sc_mesh,
    compiler_params=pltpu.CompilerParams(
        use_tc_tiling_on_sc=False, needs_layout_passes=False),
    scratch_types=[
        pltpu.VMEM((POS_W + 8,), jnp.int32),
        pltpu.VMEM((POS_W, CHUNK), jnp.int32),
        pltpu.VMEM((POS_W, CHUNK), jnp.float32),
        pltpu.SemaphoreType.DMA,
    ],
)
def _sc_gather(tblf_hbm, cids_hbm, out_hbm, buf_v, ids_v, rows_v, sem):
    cid = lax.axis_index("c")
    sid = lax.axis_index("s")
    wid = sid * 2 + cid
    base = wid * POS_W

    @pl.when(wid > 0)
    def _():
        pltpu.sync_copy(cids_hbm.at[pl.ds(base - 8, POS_W + 8)], buf_v)

    @pl.when(wid == 0)
    def _():
        pltpu.sync_copy(cids_hbm.at[pl.ds(0, POS_W)], buf_v.at[pl.ds(8, POS_W)])

    seq_start = base % N == 0
    lane = lax.iota(jnp.int32, 16)
    half = lane >> 3
    d_off = (lane & 7) * 128
    mv, pv_, cv = [], [], []
    for v in range(8):
        h = 2 * v + half
        mv.append(h + 1)
        pv_.append(jnp.where(half == 0, int(PRIMES_I32[2 * v]),
                             int(PRIMES_I32[2 * v + 1])))
        cv.append(h * (1536 * 1024) + d_off)

    def compute(p, carry):
        cur = plsc.load_gather(buf_v, [jnp.full((16,), 8, jnp.int32) + p])
        prev = plsc.load_gather(buf_v, [jnp.full((16,), 7, jnp.int32) + p])
        prev = jnp.where(
            jnp.full((16,), jnp.logical_and(seq_start, p == 0)), 0, prev)
        pair = cur + prev * UNIGRAM_VOCAB
        for v in range(8):
            b = (pair * mv[v] + mv[v]) % pv_[v]
            b = jnp.where(b >= NGRAM_VOCAB, b - NGRAM_VOCAB, b)
            w = ((b >> 7) << 10) + (b & 127)
            ids_v[p, pl.ds(v * 16, 16)] = w + cv[v]
        return carry

    lax.fori_loop(0, POS_W, compute, 0)

    def start(j):
        return pltpu.make_async_copy(
            tblf_hbm.at[ids_v.at[j]], rows_v.at[j], sem)

    start(0).start()
    start(1).start()
    start(2).start()

    def gather_chunk(j, carry):
        @pl.when(j < POS_W - 3)
        def _():
            start(j + 3).start()
        start(j).wait()
        return carry

    lax.fori_loop(0, POS_W, gather_chunk, 0)

    pltpu.sync_copy(rows_v, out_hbm.at[pl.ds(base, POS_W)])


_heads64 = np.arange(MODEL) // DIM_PER_HEAD
S64 = (_heads64[:, None] == np.arange(NUM_HEADS)[None, :]).astype(np.float32)
S64T = np.ascontiguousarray(S64.T)
_heads8 = np.arange(NGFLAT) // NGRAM_EMB_DIM
S8 = (_heads8[:, None] == np.arange(NUM_HEADS)[None, :]).astype(np.float32)
S8T = np.ascontiguousarray(S8.T)
PLACE = np.zeros((NGFLAT, MODEL), dtype=np.float32)
for _j in range(NGFLAT):
    PLACE[_j, (_j // NGRAM_EMB_DIM) * DIM_PER_HEAD +
          (DIM_PER_HEAD - NGRAM_EMB_DIM) + (_j % NGRAM_EMB_DIM)] = 1.0
KEEP = ((np.arange(MODEL) % DIM_PER_HEAD < DIM_PER_HEAD - NGRAM_EMB_DIM)
        .astype(np.float32)[None, :])

ROWS_BLK = 512


def _tc_body(x_ref, ng_ref, s64_ref, s64t_ref, s8_ref, s8t_ref, p_ref,
             ge_ref, be_ref, gn_ref, bn_ref, o_ref):
    x = x_ref[...]
    s64 = s64_ref[...]
    s64t = s64t_ref[...]
    mean = lax.dot(x, s64) * (1.0 / DIM_PER_HEAD)
    xc = x - lax.dot(mean, s64t)
    var = lax.dot(xc * xc, s64) * (1.0 / DIM_PER_HEAD)
    inv = 1.0 / (jnp.sqrt(var) + EPS)
    part = xc * lax.dot(inv, s64t) * ge_ref[...] + be_ref[...]

    ng = ng_ref[...]
    s8 = s8_ref[...]
    s8t = s8t_ref[...]
    m8 = lax.dot(ng, s8) * (1.0 / NGRAM_EMB_DIM)
    nc = ng - lax.dot(m8, s8t)
    v8 = lax.dot(nc * nc, s8) * (1.0 / NGRAM_EMB_DIM)
    inv8 = 1.0 / (jnp.sqrt(v8) + EPS)
    nn = nc * lax.dot(inv8, s8t) * gn_ref[...] + bn_ref[...]

    o_ref[...] = part + lax.dot(nn, p_ref[...])


_tc_norm = pl.pallas_call(
    _tc_body,
    grid=(POS // ROWS_BLK,),
    in_specs=[
        pl.BlockSpec((ROWS_BLK, MODEL), lambda i: (i, 0)),
        pl.BlockSpec((ROWS_BLK, NGFLAT), lambda i: (i, 0)),
        pl.BlockSpec((MODEL, NUM_HEADS), lambda i: (0, 0)),
        pl.BlockSpec((NUM_HEADS, MODEL), lambda i: (0, 0)),
        pl.BlockSpec((NGFLAT, NUM_HEADS), lambda i: (0, 0)),
        pl.BlockSpec((NUM_HEADS, NGFLAT), lambda i: (0, 0)),
        pl.BlockSpec((NGFLAT, MODEL), lambda i: (0, 0)),
        pl.BlockSpec((1, MODEL), lambda i: (0, 0)),
        pl.BlockSpec((1, MODEL), lambda i: (0, 0)),
        pl.BlockSpec((1, NGFLAT), lambda i: (0, 0)),
        pl.BlockSpec((1, NGFLAT), lambda i: (0, 0)),
    ],
    out_specs=pl.BlockSpec((ROWS_BLK, MODEL), lambda i: (i, 0)),
    out_shape=jax.ShapeDtypeStruct((POS, MODEL), jnp.float32),
)


def kernel(embeds, cluster_ids, table, g_ngram, b_ngram, g_emb, b_emb):
    b, n, d = embeds.shape
    cids = cluster_ids.reshape(-1).astype(jnp.int32)
    tblf = table.T.reshape(NGRAM_EMB_DIM, -1, 128).transpose(1, 0, 2).reshape(-1)
    ng = _sc_gather(tblf, cids)
    x = embeds.reshape(POS, MODEL)
    ge = g_emb.reshape(1, MODEL) * KEEP
    be = b_emb.reshape(1, MODEL) * KEEP
    gn = g_ngram.reshape(1, NGFLAT)
    bn = b_ngram.reshape(1, NGFLAT)
    f32 = jnp.float32
    out = _tc_norm(x, ng, jnp.asarray(S64, f32), jnp.asarray(S64T, f32),
                   jnp.asarray(S8, f32), jnp.asarray(S8T, f32),
                   jnp.asarray(PLACE, f32), ge, be, gn, bn)
    return out.reshape(b, n, d)


# speedup vs baseline: 7.7787x; 1.1664x over previous
import functools
import numpy as np
import jax
import jax.numpy as jnp
from jax import lax
from jax.experimental import pallas as pl
from jax.experimental.pallas import tpu as pltpu
from jax.experimental.pallas import tpu_sc as plsc

UNIGRAM_VOCAB = 1024
NGRAM_VOCAB = 768 * 256
NUM_HEADS = 16
DIM_PER_HEAD = 64
NGRAM_EMB_DIM = 8
EPS = 1e-05

B = 4
N = 2048
POS = B * N
MODEL = NUM_HEADS * DIM_PER_HEAD
NGFLAT = NUM_HEADS * NGRAM_EMB_DIM

NW = 32
POS_W = POS // NW
CHUNK = NGFLAT


def _hash_primes(n, count):
    primes = []
    c = n + 1
    while len(primes) < count:
        k, is_p = 2, True
        while k * k <= c:
            if c % k == 0:
                is_p = False
                break
            k += 1
        if is_p:
            primes.append(c)
        c += 1
    return primes


PRIMES_I32 = np.array(_hash_primes(NGRAM_VOCAB, NUM_HEADS), dtype=np.int32)


_sc_mesh = plsc.VectorSubcoreMesh(core_axis_name="c", subcore_axis_name="s")


@functools.partial(
    pl.kernel,
    out_type=jax.ShapeDtypeStruct((NW * 32, 8 * CHUNK), jnp.float32),
    mesh=_sc_mesh,
    compiler_params=pltpu.CompilerParams(
        use_tc_tiling_on_sc=False, needs_layout_passes=False),
    scratch_types=[
        pltpu.VMEM((POS_W + 8,), jnp.int32),
        pltpu.VMEM((32, 8 * CHUNK), jnp.int32),
        pltpu.VMEM((32, 8 * CHUNK), jnp.float32),
        pltpu.SemaphoreType.DMA,
    ],
)
def _sc_gather(tblf_hbm, cids_hbm, out_hbm, buf_v, ids_v, rows_v, sem):
    cid = lax.axis_index("c")
    sid = lax.axis_index("s")
    wid = sid * 2 + cid
    base = wid * POS_W

    @pl.when(wid > 0)
    def _():
        pltpu.sync_copy(cids_hbm.at[pl.ds(base - 8, POS_W + 8)], buf_v)

    @pl.when(wid == 0)
    def _():
        pltpu.sync_copy(cids_hbm.at[pl.ds(0, POS_W)], buf_v.at[pl.ds(8, POS_W)])

    seq_start = base % N == 0
    lane = lax.iota(jnp.int32, 16)
    half = lane >> 3
    d_off = (lane & 7) * 128
    mv, pv_, cv = [], [], []
    for v in range(8):
        h = 2 * v + half
        mv.append(h + 1)
        pv_.append(jnp.where(half == 0, int(PRIMES_I32[2 * v]),
                             int(PRIMES_I32[2 * v + 1])))
        cv.append(h * (1536 * 1024) + d_off)

    def compute(p, carry):
        cur = plsc.load_gather(buf_v, [jnp.full((16,), 8, jnp.int32) + p])
        prev = plsc.load_gather(buf_v, [jnp.full((16,), 7, jnp.int32) + p])
        prev = jnp.where(
            jnp.full((16,), jnp.logical_and(seq_start, p == 0)), 0, prev)
        pair = cur + prev * UNIGRAM_VOCAB
        for v in range(8):
            b = (pair * mv[v] + mv[v]) % pv_[v]
            b = jnp.where(b >= NGRAM_VOCAB, b - NGRAM_VOCAB, b)
            w = ((b >> 7) << 10) + (b & 127)
            ids_v[p // 8, pl.ds((p % 8) * CHUNK + v * 16, 16)] = w + cv[v]
        return carry

    lax.fori_loop(0, POS_W, compute, 0)

    def start(j):
        return pltpu.make_async_copy(
            tblf_hbm.at[ids_v.at[j]], rows_v.at[j], sem)

    start(0).start()
    start(1).start()
    start(2).start()

    def gather_chunk(j, carry):
        @pl.when(j < 32 - 3)
        def _():
            start(j + 3).start()
        start(j).wait()
        return carry

    lax.fori_loop(0, 32, gather_chunk, 0)

    pltpu.sync_copy(rows_v, out_hbm.at[pl.ds(wid * 32, 32)])


_heads64 = np.arange(MODEL) // DIM_PER_HEAD
S64 = (_heads64[:, None] == np.arange(NUM_HEADS)[None, :]).astype(np.float32)
S64T = np.ascontiguousarray(S64.T)
_heads8 = np.arange(NGFLAT) // NGRAM_EMB_DIM
S8 = (_heads8[:, None] == np.arange(NUM_HEADS)[None, :]).astype(np.float32)
S8T = np.ascontiguousarray(S8.T)
PLACE = np.zeros((NGFLAT, MODEL), dtype=np.float32)
for _j in range(NGFLAT):
    PLACE[_j, (_j // NGRAM_EMB_DIM) * DIM_PER_HEAD +
          (DIM_PER_HEAD - NGRAM_EMB_DIM) + (_j % NGRAM_EMB_DIM)] = 1.0
KEEP = ((np.arange(MODEL) % DIM_PER_HEAD < DIM_PER_HEAD - NGRAM_EMB_DIM)
        .astype(np.float32)[None, :])

ROWS_BLK = 512


def _tc_body(x_ref, ng_ref, s64_ref, s64t_ref, s8_ref, s8t_ref, p_ref,
             ge_ref, be_ref, gn_ref, bn_ref, o_ref):
    x = x_ref[...]
    s64 = s64_ref[...]
    s64t = s64t_ref[...]
    mean = lax.dot(x, s64) * (1.0 / DIM_PER_HEAD)
    xc = x - lax.dot(mean, s64t)
    var = lax.dot(xc * xc, s64) * (1.0 / DIM_PER_HEAD)
    inv = 1.0 / (jnp.sqrt(var) + EPS)
    part = xc * lax.dot(inv, s64t) * ge_ref[...] + be_ref[...]

    ng = ng_ref[...]
    s8 = s8_ref[...]
    s8t = s8t_ref[...]
    m8 = lax.dot(ng, s8) * (1.0 / NGRAM_EMB_DIM)
    nc = ng - lax.dot(m8, s8t)
    v8 = lax.dot(nc * nc, s8) * (1.0 / NGRAM_EMB_DIM)
    inv8 = 1.0 / (jnp.sqrt(v8) + EPS)
    nn = nc * lax.dot(inv8, s8t) * gn_ref[...] + bn_ref[...]

    o_ref[...] = part + lax.dot(nn, p_ref[...])


_tc_norm = pl.pallas_call(
    _tc_body,
    grid=(POS // ROWS_BLK,),
    in_specs=[
        pl.BlockSpec((ROWS_BLK, MODEL), lambda i: (i, 0)),
        pl.BlockSpec((ROWS_BLK, NGFLAT), lambda i: (i, 0)),
        pl.BlockSpec((MODEL, NUM_HEADS), lambda i: (0, 0)),
        pl.BlockSpec((NUM_HEADS, MODEL), lambda i: (0, 0)),
        pl.BlockSpec((NGFLAT, NUM_HEADS), lambda i: (0, 0)),
        pl.BlockSpec((NUM_HEADS, NGFLAT), lambda i: (0, 0)),
        pl.BlockSpec((NGFLAT, MODEL), lambda i: (0, 0)),
        pl.BlockSpec((1, MODEL), lambda i: (0, 0)),
        pl.BlockSpec((1, MODEL), lambda i: (0, 0)),
        pl.BlockSpec((1, NGFLAT), lambda i: (0, 0)),
        pl.BlockSpec((1, NGFLAT), lambda i: (0, 0)),
    ],
    out_specs=pl.BlockSpec((ROWS_BLK, MODEL), lambda i: (i, 0)),
    out_shape=jax.ShapeDtypeStruct((POS, MODEL), jnp.float32),
)


def kernel(embeds, cluster_ids, table, g_ngram, b_ngram, g_emb, b_emb):
    b, n, d = embeds.shape
    cids = cluster_ids.reshape(-1).astype(jnp.int32)
    tblf = table.T.reshape(NGRAM_EMB_DIM, -1, 128).transpose(1, 0, 2).reshape(-1)
    ng = _sc_gather(tblf, cids).reshape(POS, NGFLAT)
    x = embeds.reshape(POS, MODEL)
    ge = g_emb.reshape(1, MODEL) * KEEP
    be = b_emb.reshape(1, MODEL) * KEEP
    gn = g_ngram.reshape(1, NGFLAT)
    bn = b_ngram.reshape(1, NGFLAT)
    f32 = jnp.float32
    out = _tc_norm(x, ng, jnp.asarray(S64, f32), jnp.asarray(S64T, f32),
                   jnp.asarray(S8, f32), jnp.asarray(S8T, f32),
                   jnp.asarray(PLACE, f32), ge, be, gn, bn)
    return out.reshape(b, n, d)


# speedup vs baseline: 12.2473x; 1.5745x over previous
import functools
import numpy as np
import jax
import jax.numpy as jnp
from jax import lax
from jax.experimental import pallas as pl
from jax.experimental.pallas import tpu as pltpu
from jax.experimental.pallas import tpu_sc as plsc

UNIGRAM_VOCAB = 1024
NGRAM_VOCAB = 768 * 256
NUM_HEADS = 16
DIM_PER_HEAD = 64
NGRAM_EMB_DIM = 8
EPS = 1e-05

B = 4
N = 2048
POS = B * N
MODEL = NUM_HEADS * DIM_PER_HEAD
NGFLAT = NUM_HEADS * NGRAM_EMB_DIM

NW = 32
POS_W = POS // NW
CHUNK = NGFLAT


def _hash_primes(n, count):
    primes = []
    c = n + 1
    while len(primes) < count:
        k, is_p = 2, True
        while k * k <= c:
            if c % k == 0:
                is_p = False
                break
            k += 1
        if is_p:
            primes.append(c)
        c += 1
    return primes


PRIMES_I32 = np.array(_hash_primes(NGRAM_VOCAB, NUM_HEADS), dtype=np.int32)


_sc_mesh = plsc.VectorSubcoreMesh(core_axis_name="c", subcore_axis_name="s")


@functools.partial(
    pl.kernel,
    out_type=jax.ShapeDtypeStruct((NW * 32, 8 * CHUNK), jnp.float32),
    mesh=_sc_mesh,
    compiler_params=pltpu.CompilerParams(
        use_tc_tiling_on_sc=False, needs_layout_passes=False),
    scratch_types=[
        pltpu.VMEM((POS_W + 8,), jnp.int32),
        pltpu.VMEM((32, 8 * CHUNK), jnp.int32),
        pltpu.VMEM((32, 8 * CHUNK), jnp.float32),
        pltpu.SemaphoreType.DMA,
    ],
)
def _sc_gather(tblf_hbm, cids_hbm, out_hbm, buf_v, ids_v, rows_v, sem):
    cid = lax.axis_index("c")
    sid = lax.axis_index("s")
    wid = sid * 2 + cid
    base = wid * POS_W

    @pl.when(wid > 0)
    def _():
        pltpu.sync_copy(cids_hbm.at[pl.ds(base - 8, POS_W + 8)], buf_v)

    @pl.when(wid == 0)
    def _():
        pltpu.sync_copy(cids_hbm.at[pl.ds(0, POS_W)], buf_v.at[pl.ds(8, POS_W)])

    seq_start = base % N == 0
    lane = lax.iota(jnp.int32, 16)
    half = lane >> 3
    d_off = (lane & 7) * 128
    mv, pv_, inv_, cv = [], [], [], []
    for v in range(8):
        h = 2 * v + half
        p0, p1 = int(PRIMES_I32[2 * v]), int(PRIMES_I32[2 * v + 1])
        mv.append(h + 1)
        pv_.append(jnp.where(half == 0, p0, p1))
        inv_.append(jnp.where(half == 0, jnp.float32(1.0 / p0),
                              jnp.float32(1.0 / p1)))
        cv.append(h * (1536 * 1024) + d_off)

    def start(j):
        return pltpu.make_async_copy(
            tblf_hbm.at[ids_v.at[j]], rows_v.at[j], sem)

    def chunk_body(j, carry):
        for q in range(8):
            p = j * 8 + q
            cur = plsc.load_gather(buf_v, [jnp.full((16,), 8, jnp.int32) + p])
            prev = plsc.load_gather(buf_v, [jnp.full((16,), 7, jnp.int32) + p])
            prev = jnp.where(
                jnp.full((16,), jnp.logical_and(seq_start, p == 0)), 0, prev)
            pair = cur + prev * UNIGRAM_VOCAB
            for v in range(8):
                x = pair * mv[v] + mv[v]
                q_ = (x.astype(jnp.float32) * inv_[v]).astype(jnp.int32)
                b = x - q_ * pv_[v]
                b = jnp.where(b < 0, b + pv_[v], b)
                b = jnp.where(b >= pv_[v], b - pv_[v], b)
                b = jnp.where(b >= NGRAM_VOCAB, b - NGRAM_VOCAB, b)
                w = ((b >> 7) << 10) + (b & 127)
                ids_v[j, pl.ds(q * CHUNK + v * 16, 16)] = w + cv[v]
        start(j).start()

        @pl.when(j >= 2)
        def _():
            start(j - 2).wait()
        return carry

    lax.fori_loop(0, 32, chunk_body, 0)
    start(30).wait()
    start(31).wait()

    pltpu.sync_copy(rows_v, out_hbm.at[pl.ds(wid * 32, 32)])


_heads64 = np.arange(MODEL) // DIM_PER_HEAD
S64 = (_heads64[:, None] == np.arange(NUM_HEADS)[None, :]).astype(np.float32)
S64T = np.ascontiguousarray(S64.T)
_heads8 = np.arange(NGFLAT) // NGRAM_EMB_DIM
S8 = (_heads8[:, None] == np.arange(NUM_HEADS)[None, :]).astype(np.float32)
S8T = np.ascontiguousarray(S8.T)
PLACE = np.zeros((NGFLAT, MODEL), dtype=np.float32)
for _j in range(NGFLAT):
    PLACE[_j, (_j // NGRAM_EMB_DIM) * DIM_PER_HEAD +
          (DIM_PER_HEAD - NGRAM_EMB_DIM) + (_j % NGRAM_EMB_DIM)] = 1.0
KEEP = ((np.arange(MODEL) % DIM_PER_HEAD < DIM_PER_HEAD - NGRAM_EMB_DIM)
        .astype(np.float32)[None, :])

ROWS_BLK = 512


def _tc_body(x_ref, ng_ref, s64_ref, s64t_ref, s8_ref, s8t_ref, p_ref,
             ge_ref, be_ref, gn_ref, bn_ref, o_ref):
    x = x_ref[...]
    s64 = s64_ref[...]
    s64t = s64t_ref[...]
    mean = lax.dot(x, s64) * (1.0 / DIM_PER_HEAD)
    xc = x - lax.dot(mean, s64t)
    var = lax.dot(xc * xc, s64) * (1.0 / DIM_PER_HEAD)
    inv = 1.0 / (jnp.sqrt(var) + EPS)
    part = xc * lax.dot(inv, s64t) * ge_ref[...] + be_ref[...]

    ng = ng_ref[...]
    s8 = s8_ref[...]
    s8t = s8t_ref[...]
    m8 = lax.dot(ng, s8) * (1.0 / NGRAM_EMB_DIM)
    nc = ng - lax.dot(m8, s8t)
    v8 = lax.dot(nc * nc, s8) * (1.0 / NGRAM_EMB_DIM)
    inv8 = 1.0 / (jnp.sqrt(v8) + EPS)
    nn = nc * lax.dot(inv8, s8t) * gn_ref[...] + bn_ref[...]

    o_ref[...] = part + lax.dot(nn, p_ref[...])


_tc_norm = pl.pallas_call(
    _tc_body,
    grid=(POS // ROWS_BLK,),
    in_specs=[
        pl.BlockSpec((ROWS_BLK, MODEL), lambda i: (i, 0)),
        pl.BlockSpec((ROWS_BLK, NGFLAT), lambda i: (i, 0)),
        pl.BlockSpec((MODEL, NUM_HEADS), lambda i: (0, 0)),
        pl.BlockSpec((NUM_HEADS, MODEL), lambda i: (0, 0)),
        pl.BlockSpec((NGFLAT, NUM_HEADS), lambda i: (0, 0)),
        pl.BlockSpec((NUM_HEADS, NGFLAT), lambda i: (0, 0)),
        pl.BlockSpec((NGFLAT, MODEL), lambda i: (0, 0)),
        pl.BlockSpec((1, MODEL), lambda i: (0, 0)),
        pl.BlockSpec((1, MODEL), lambda i: (0, 0)),
        pl.BlockSpec((1, NGFLAT), lambda i: (0, 0)),
        pl.BlockSpec((1, NGFLAT), lambda i: (0, 0)),
    ],
    out_specs=pl.BlockSpec((ROWS_BLK, MODEL), lambda i: (i, 0)),
    out_shape=jax.ShapeDtypeStruct((POS, MODEL), jnp.float32),
)


def kernel(embeds, cluster_ids, table, g_ngram, b_ngram, g_emb, b_emb):
    b, n, d = embeds.shape
    cids = cluster_ids.reshape(-1).astype(jnp.int32)
    tblf = table.T.reshape(NGRAM_EMB_DIM, -1, 128).transpose(1, 0, 2).reshape(-1)
    ng = _sc_gather(tblf, cids).reshape(POS, NGFLAT)
    x = embeds.reshape(POS, MODEL)
    ge = g_emb.reshape(1, MODEL) * KEEP
    be = b_emb.reshape(1, MODEL) * KEEP
    gn = g_ngram.reshape(1, NGFLAT)
    bn = b_ngram.reshape(1, NGFLAT)
    f32 = jnp.float32
    out = _tc_norm(x, ng, jnp.asarray(S64, f32), jnp.asarray(S64T, f32),
                   jnp.asarray(S8, f32), jnp.asarray(S8T, f32),
                   jnp.asarray(PLACE, f32), ge, be, gn, bn)
    return out.reshape(b, n, d)


# speedup vs baseline: 12.4755x; 1.0186x over previous
import functools
import numpy as np
import jax
import jax.numpy as jnp
from jax import lax
from jax.experimental import pallas as pl
from jax.experimental.pallas import tpu as pltpu
from jax.experimental.pallas import tpu_sc as plsc

UNIGRAM_VOCAB = 1024
NGRAM_VOCAB = 768 * 256
NUM_HEADS = 16
DIM_PER_HEAD = 64
NGRAM_EMB_DIM = 8
EPS = 1e-05

B = 4
N = 2048
POS = B * N
MODEL = NUM_HEADS * DIM_PER_HEAD
NGFLAT = NUM_HEADS * NGRAM_EMB_DIM

NW = 32
POS_W = POS // NW
CHUNK = NGFLAT


def _hash_primes(n, count):
    primes = []
    c = n + 1
    while len(primes) < count:
        k, is_p = 2, True
        while k * k <= c:
            if c % k == 0:
                is_p = False
                break
            k += 1
        if is_p:
            primes.append(c)
        c += 1
    return primes


PRIMES_I32 = np.array(_hash_primes(NGRAM_VOCAB, NUM_HEADS), dtype=np.int32)


_sc_mesh = plsc.VectorSubcoreMesh(core_axis_name="c", subcore_axis_name="s")


@functools.partial(
    pl.kernel,
    out_type=jax.ShapeDtypeStruct((NW * 16, 16 * CHUNK), jnp.float32),
    mesh=_sc_mesh,
    compiler_params=pltpu.CompilerParams(
        use_tc_tiling_on_sc=False, needs_layout_passes=False),
    scratch_types=[
        pltpu.VMEM((POS_W + 8,), jnp.int32),
        pltpu.VMEM((16, 16 * CHUNK), jnp.int32),
        pltpu.VMEM((16, 16 * CHUNK), jnp.float32),
        pltpu.SemaphoreType.DMA,
    ],
)
def _sc_gather(tblf_hbm, cids_hbm, out_hbm, buf_v, ids_v, rows_v, sem):
    cid = lax.axis_index("c")
    sid = lax.axis_index("s")
    wid = sid * 2 + cid
    base = wid * POS_W

    @pl.when(wid > 0)
    def _():
        pltpu.sync_copy(cids_hbm.at[pl.ds(base - 8, POS_W + 8)], buf_v)

    @pl.when(wid == 0)
    def _():
        pltpu.sync_copy(cids_hbm.at[pl.ds(0, POS_W)], buf_v.at[pl.ds(8, POS_W)])

    seq_start = base % N == 0
    lane = lax.iota(jnp.int32, 16)
    half = lane >> 3
    d_off = (lane & 7) * 128
    mv, pv_, inv_, cv = [], [], [], []
    for v in range(8):
        h = 2 * v + half
        p0, p1 = int(PRIMES_I32[2 * v]), int(PRIMES_I32[2 * v + 1])
        mv.append(h + 1)
        pv_.append(jnp.where(half == 0, p0, p1))
        inv_.append(jnp.where(half == 0, jnp.float32(1.0 / p0),
                              jnp.float32(1.0 / p1)))
        cv.append(h * (1536 * 1024) + d_off)

    def start(j):
        return pltpu.make_async_copy(
            tblf_hbm.at[ids_v.at[j]], rows_v.at[j], sem)

    def chunk_body(j, carry):
        for q in range(16):
            p = j * 16 + q
            cur = plsc.load_gather(buf_v, [jnp.full((16,), 8, jnp.int32) + p])
            prev = plsc.load_gather(buf_v, [jnp.full((16,), 7, jnp.int32) + p])
            prev = jnp.where(
                jnp.full((16,), jnp.logical_and(seq_start, p == 0)), 0, prev)
            pair = cur + prev * UNIGRAM_VOCAB
            for v in range(8):
                x = pair * mv[v] + mv[v]
                q_ = (x.astype(jnp.float32) * inv_[v]).astype(jnp.int32)
                b = x - q_ * pv_[v]
                b = jnp.where(b < 0, b + pv_[v], b)
                b = jnp.where(b >= pv_[v], b - pv_[v], b)
                b = jnp.where(b >= NGRAM_VOCAB, b - NGRAM_VOCAB, b)
                w = ((b >> 7) << 10) + (b & 127)
                ids_v[j, pl.ds(q * CHUNK + v * 16, 16)] = w + cv[v]
        start(j).start()

        @pl.when(j >= 2)
        def _():
            start(j - 2).wait()
        return carry

    lax.fori_loop(0, 16, chunk_body, 0)
    start(14).wait()
    start(15).wait()

    pltpu.sync_copy(rows_v, out_hbm.at[pl.ds(wid * 16, 16)])


_heads64 = np.arange(MODEL) // DIM_PER_HEAD
S64 = (_heads64[:, None] == np.arange(NUM_HEADS)[None, :]).astype(np.float32)
S64T = np.ascontiguousarray(S64.T)
_heads8 = np.arange(NGFLAT) // NGRAM_EMB_DIM
S8 = (_heads8[:, None] == np.arange(NUM_HEADS)[None, :]).astype(np.float32)
S8T = np.ascontiguousarray(S8.T)
PLACE = np.zeros((NGFLAT, MODEL), dtype=np.float32)
for _j in range(NGFLAT):
    PLACE[_j, (_j // NGRAM_EMB_DIM) * DIM_PER_HEAD +
          (DIM_PER_HEAD - NGRAM_EMB_DIM) + (_j % NGRAM_EMB_DIM)] = 1.0
KEEP = ((np.arange(MODEL) % DIM_PER_HEAD < DIM_PER_HEAD - NGRAM_EMB_DIM)
        .astype(np.float32)[None, :])

ROWS_BLK = 512


def _tc_body(x_ref, ng_ref, s64_ref, s64t_ref, s8_ref, s8t_ref, p_ref,
             ge_ref, be_ref, gn_ref, bn_ref, o_ref):
    x = x_ref[...]
    s64 = s64_ref[...]
    s64t = s64t_ref[...]
    mean = lax.dot(x, s64) * (1.0 / DIM_PER_HEAD)
    xc = x - lax.dot(mean, s64t)
    var = lax.dot(xc * xc, s64) * (1.0 / DIM_PER_HEAD)
    inv = 1.0 / (jnp.sqrt(var) + EPS)
    part = xc * lax.dot(inv, s64t) * ge_ref[...] + be_ref[...]

    ng = ng_ref[...]
    s8 = s8_ref[...]
    s8t = s8t_ref[...]
    m8 = lax.dot(ng, s8) * (1.0 / NGRAM_EMB_DIM)
    nc = ng - lax.dot(m8, s8t)
    v8 = lax.dot(nc * nc, s8) * (1.0 / NGRAM_EMB_DIM)
    inv8 = 1.0 / (jnp.sqrt(v8) + EPS)
    nn = nc * lax.dot(inv8, s8t) * gn_ref[...] + bn_ref[...]

    o_ref[...] = part + lax.dot(nn, p_ref[...])


_tc_norm = pl.pallas_call(
    _tc_body,
    grid=(POS // ROWS_BLK,),
    in_specs=[
        pl.BlockSpec((ROWS_BLK, MODEL), lambda i: (i, 0)),
        pl.BlockSpec((ROWS_BLK, NGFLAT), lambda i: (i, 0)),
        pl.BlockSpec((MODEL, NUM_HEADS), lambda i: (0, 0)),
        pl.BlockSpec((NUM_HEADS, MODEL), lambda i: (0, 0)),
        pl.BlockSpec((NGFLAT, NUM_HEADS), lambda i: (0, 0)),
        pl.BlockSpec((NUM_HEADS, NGFLAT), lambda i: (0, 0)),
        pl.BlockSpec((NGFLAT, MODEL), lambda i: (0, 0)),
        pl.BlockSpec((1, MODEL), lambda i: (0, 0)),
        pl.BlockSpec((1, MODEL), lambda i: (0, 0)),
        pl.BlockSpec((1, NGFLAT), lambda i: (0, 0)),
        pl.BlockSpec((1, NGFLAT), lambda i: (0, 0)),
    ],
    out_specs=pl.BlockSpec((ROWS_BLK, MODEL), lambda i: (i, 0)),
    out_shape=jax.ShapeDtypeStruct((POS, MODEL), jnp.float32),
)


def kernel(embeds, cluster_ids, table, g_ngram, b_ngram, g_emb, b_emb):
    b, n, d = embeds.shape
    cids = cluster_ids.reshape(-1).astype(jnp.int32)
    tblf = table.T.reshape(NGRAM_EMB_DIM, -1, 128).transpose(1, 0, 2).reshape(-1)
    ng = _sc_gather(tblf, cids).reshape(POS, NGFLAT)
    x = embeds.reshape(POS, MODEL)
    ge = g_emb.reshape(1, MODEL) * KEEP
    be = b_emb.reshape(1, MODEL) * KEEP
    gn = g_ngram.reshape(1, NGFLAT)
    bn = b_ngram.reshape(1, NGFLAT)
    f32 = jnp.float32
    out = _tc_norm(x, ng, jnp.asarray(S64, f32), jnp.asarray(S64T, f32),
                   jnp.asarray(S8, f32), jnp.asarray(S8T, f32),
                   jnp.asarray(PLACE, f32), ge, be, gn, bn)
    return out.reshape(b, n, d)


# speedup vs baseline: 14.6906x; 1.1775x over previous
import functools
import numpy as np
import jax
import jax.numpy as jnp
from jax import lax
from jax.experimental import pallas as pl
from jax.experimental.pallas import tpu as pltpu
from jax.experimental.pallas import tpu_sc as plsc

UNIGRAM_VOCAB = 1024
NGRAM_VOCAB = 768 * 256
NUM_HEADS = 16
DIM_PER_HEAD = 64
NGRAM_EMB_DIM = 8
EPS = 1e-05

B = 4
N = 2048
POS = B * N
MODEL = NUM_HEADS * DIM_PER_HEAD
NGFLAT = NUM_HEADS * NGRAM_EMB_DIM

NW = 32
POS_W = POS // NW
CHUNK = NGFLAT


def _hash_primes(n, count):
    primes = []
    c = n + 1
    while len(primes) < count:
        k, is_p = 2, True
        while k * k <= c:
            if c % k == 0:
                is_p = False
                break
            k += 1
        if is_p:
            primes.append(c)
        c += 1
    return primes


PRIMES_I32 = np.array(_hash_primes(NGRAM_VOCAB, NUM_HEADS), dtype=np.int32)


_sc_mesh = plsc.VectorSubcoreMesh(core_axis_name="c", subcore_axis_name="s")


@functools.partial(
    pl.kernel,
    out_type=jax.ShapeDtypeStruct((NW * 16, 16 * CHUNK), jnp.float32),
    mesh=_sc_mesh,
    compiler_params=pltpu.CompilerParams(
        use_tc_tiling_on_sc=False, needs_layout_passes=False),
    scratch_types=[
        pltpu.VMEM((POS_W + 8,), jnp.int32),
        pltpu.VMEM((16, 16 * CHUNK), jnp.int32),
        pltpu.VMEM((16, 16 * CHUNK), jnp.float32),
        pltpu.SemaphoreType.DMA,
    ],
)
def _sc_gather(tblf_hbm, cids_hbm, out_hbm, buf_v, ids_v, rows_v, sem):
    cid = lax.axis_index("c")
    sid = lax.axis_index("s")
    wid = sid * 2 + cid
    base = wid * POS_W

    @pl.when(wid > 0)
    def _():
        pltpu.sync_copy(cids_hbm.at[pl.ds(base - 8, POS_W + 8)], buf_v)

    @pl.when(wid == 0)
    def _():
        pltpu.sync_copy(cids_hbm.at[pl.ds(0, POS_W)], buf_v.at[pl.ds(8, POS_W)])

    seq_start = base % N == 0
    lane = lax.iota(jnp.int32, 16)
    half = lane >> 3
    d_off = (lane & 7) * 128
    mv, pv_, inv_, cv = [], [], [], []
    for v in range(8):
        h = 2 * v + half
        p0, p1 = int(PRIMES_I32[2 * v]), int(PRIMES_I32[2 * v + 1])
        mv.append(h + 1)
        pv_.append(jnp.where(half == 0, p0, p1))
        inv_.append(jnp.where(half == 0, jnp.float32(1.0 / p0),
                              jnp.float32(1.0 / p1)))
        cv.append(h * (1536 * 1024) + d_off)

    def start(j):
        return pltpu.make_async_copy(
            tblf_hbm.at[ids_v.at[j]], rows_v.at[j], sem)

    def chunk_body(j, carry):
        for q in range(16):
            p = j * 16 + q
            cur = plsc.load_gather(buf_v, [jnp.full((16,), 8, jnp.int32) + p])
            prev = plsc.load_gather(buf_v, [jnp.full((16,), 7, jnp.int32) + p])
            prev = jnp.where(
                jnp.full((16,), jnp.logical_and(seq_start, p == 0)), 0, prev)
            pair = cur + prev * UNIGRAM_VOCAB
            for v in range(8):
                x = pair * mv[v] + mv[v]
                q_ = (x.astype(jnp.float32) * inv_[v]).astype(jnp.int32)
                b = x - q_ * pv_[v]
                b = jnp.where(b < 0, b + pv_[v], b)
                b = jnp.where(b >= pv_[v], b - pv_[v], b)
                b = jnp.where(b >= NGRAM_VOCAB, b - NGRAM_VOCAB, b)
                w = ((b >> 7) << 10) + (b & 127)
                ids_v[j, pl.ds(q * CHUNK + v * 16, 16)] = w + cv[v]
        start(j).start()

        @pl.when(j >= 2)
        def _():
            start(j - 2).wait()
        return carry

    lax.fori_loop(0, 16, chunk_body, 0)
    start(14).wait()
    start(15).wait()

    pltpu.sync_copy(rows_v, out_hbm.at[pl.ds(wid * 16, 16)])


_heads64 = np.arange(MODEL) // DIM_PER_HEAD
S64 = (_heads64[:, None] == np.arange(NUM_HEADS)[None, :]).astype(np.float32)
S64T = np.ascontiguousarray(S64.T)
_heads8 = np.arange(NGFLAT) // NGRAM_EMB_DIM
S8 = (_heads8[:, None] == np.arange(NUM_HEADS)[None, :]).astype(np.float32)
S8T = np.ascontiguousarray(S8.T)
PLACE = np.zeros((NGFLAT, MODEL), dtype=np.float32)
for _j in range(NGFLAT):
    PLACE[_j, (_j // NGRAM_EMB_DIM) * DIM_PER_HEAD +
          (DIM_PER_HEAD - NGRAM_EMB_DIM) + (_j % NGRAM_EMB_DIM)] = 1.0
KEEP = ((np.arange(MODEL) % DIM_PER_HEAD < DIM_PER_HEAD - NGRAM_EMB_DIM)
        .astype(np.float32)[None, :])

ROWS_BLK = 1024


def _tc_body(x_ref, ng_ref, s64_ref, g64_ref, s8_ref, s8t_ref, pg_ref,
             bet_ref, o_ref):
    x = x_ref[...]
    s64 = s64_ref[...]
    mean = lax.dot(x, s64) * (1.0 / DIM_PER_HEAD)
    ex2 = lax.dot(x * x, s64) * (1.0 / DIM_PER_HEAD)
    var = jnp.maximum(ex2 - mean * mean, 0.0)
    inv = 1.0 / (jnp.sqrt(var) + EPS)
    g64 = g64_ref[...]
    part = x * lax.dot(inv, g64) - lax.dot(mean * inv, g64)

    ng = ng_ref[...]
    s8 = s8_ref[...]
    s8t = s8t_ref[...]
    m8 = lax.dot(ng, s8) * (1.0 / NGRAM_EMB_DIM)
    nc = ng - lax.dot(m8, s8t)
    v8 = lax.dot(nc * nc, s8) * (1.0 / NGRAM_EMB_DIM)
    inv8 = 1.0 / (jnp.sqrt(v8) + EPS)
    nn = nc * lax.dot(inv8, s8t)

    o_ref[...] = part + lax.dot(nn, pg_ref[...]) + bet_ref[...]


_tc_norm = pl.pallas_call(
    _tc_body,
    grid=(POS // ROWS_BLK,),
    in_specs=[
        pl.BlockSpec((ROWS_BLK, MODEL), lambda i: (i, 0)),
        pl.BlockSpec((ROWS_BLK, NGFLAT), lambda i: (i, 0)),
        pl.BlockSpec((MODEL, NUM_HEADS), lambda i: (0, 0)),
        pl.BlockSpec((NUM_HEADS, MODEL), lambda i: (0, 0)),
        pl.BlockSpec((NGFLAT, NUM_HEADS), lambda i: (0, 0)),
        pl.BlockSpec((NUM_HEADS, NGFLAT), lambda i: (0, 0)),
        pl.BlockSpec((NGFLAT, MODEL), lambda i: (0, 0)),
        pl.BlockSpec((1, MODEL), lambda i: (0, 0)),
    ],
    out_specs=pl.BlockSpec((ROWS_BLK, MODEL), lambda i: (i, 0)),
    out_shape=jax.ShapeDtypeStruct((POS, MODEL), jnp.float32),
)


def kernel(embeds, cluster_ids, table, g_ngram, b_ngram, g_emb, b_emb):
    b, n, d = embeds.shape
    cids = cluster_ids.reshape(-1).astype(jnp.int32)
    tblf = table.T.reshape(NGRAM_EMB_DIM, -1, 128).transpose(1, 0, 2).reshape(-1)
    ng = _sc_gather(tblf, cids).reshape(POS, NGFLAT)
    x = embeds.reshape(POS, MODEL)
    f32 = jnp.float32
    keep = jnp.asarray(KEEP, f32)
    ge = g_emb.reshape(1, MODEL) * keep
    be = b_emb.reshape(1, MODEL) * keep
    g64 = jnp.asarray(S64T, f32) * ge
    pg = jnp.asarray(PLACE, f32) * g_ngram.reshape(NGFLAT, 1)
    bet = be + b_ngram.reshape(1, NGFLAT) @ jnp.asarray(PLACE, f32)
    out = _tc_norm(x, ng, jnp.asarray(S64, f32), g64,
                   jnp.asarray(S8, f32), jnp.asarray(S8T, f32), pg, bet)
    return out.reshape(b, n, d)
